# Initial kernel scaffold; baseline (speedup 1.0000x reference)
import jax
import jax.numpy as jnp
from jax.experimental import pallas as pl


def kernel(x, edge_index, edge_weight, W1, b1, W2, b2):
    raise NotImplementedError("write your pallas kernel here")


# speedup vs baseline: 13.5348x; 13.5348x over previous
import functools
import math

import jax
import jax.numpy as jnp
from jax import lax
from jax.experimental import pallas as pl
from jax.experimental.pallas import tpu as pltpu
from jax.experimental.pallas import tpu_sc as plsc

NC = 2
NS = 16
NW = NC * NS
L = 16
CHUNK = 128


def _mesh():
    return plsc.VectorSubcoreMesh(core_axis_name="c", subcore_axis_name="s")


def _zero_rows(buf, n_rows, d):
    zv = jnp.zeros((L,), jnp.float32)

    @pl.loop(0, n_rows)
    def _(i):
        for j in range(d // L):
            buf[i, pl.ds(j * L, L)] = zv


def _make_deg_kernel(kw, n_pad):
    rpt = n_pad // NS

    @functools.partial(
        pl.kernel,
        mesh=_mesh(),
        out_type=jax.ShapeDtypeStruct((NC * n_pad,), jnp.float32),
        scratch_types=[
            pltpu.VMEM((kw, CHUNK), jnp.int32),
            pltpu.VMEM((kw, CHUNK), jnp.float32),
            pltpu.VMEM((rpt,), jnp.float32),
            pltpu.VMEM_SHARED((n_pad,), jnp.float32),
        ],
    )
    def deg_kernel(dst_hbm, w_hbm, out_hbm, dst_v, w_v, zb, acc):
        cid = lax.axis_index("c")
        sid = lax.axis_index("s")
        wid = cid * NS + sid

        @pl.loop(0, rpt // L)
        def _(i):
            zb[pl.ds(i * L, L)] = jnp.zeros((L,), jnp.float32)

        pltpu.sync_copy(zb, acc.at[pl.ds(sid * rpt, rpt)])
        plsc.subcore_barrier()

        pltpu.sync_copy(dst_hbm.at[wid], dst_v)
        pltpu.sync_copy(w_hbm.at[wid], w_v)

        @pl.loop(0, kw)
        def _(k):
            pltpu.sync_copy(w_v.at[k], acc.at[dst_v.at[k]], add=True)

        plsc.subcore_barrier()
        pltpu.sync_copy(acc.at[pl.ds(sid * rpt, rpt)],
                        out_hbm.at[pl.ds(cid * n_pad + sid * rpt, rpt)])

    return deg_kernel


def _make_agg_kernel(kw, n_pad, d, compute_norm):
    rpt = n_pad // NS

    scratch = [
        pltpu.VMEM((kw, CHUNK), jnp.int32),
        pltpu.VMEM((kw, CHUNK), jnp.int32),
        pltpu.VMEM((kw, CHUNK), jnp.float32),
        pltpu.VMEM((CHUNK, d), jnp.float32),
        pltpu.VMEM_SHARED((n_pad, d), jnp.float32),
        pltpu.SemaphoreType.DMA,
    ]
    if compute_norm:
        scratch.append(pltpu.VMEM((CHUNK,), jnp.float32))
        scratch.append(pltpu.SemaphoreType.DMA)
        out_type = (jax.ShapeDtypeStruct((NC, n_pad, d), jnp.float32),
                    jax.ShapeDtypeStruct((NW, kw, CHUNK), jnp.float32))
    else:
        out_type = jax.ShapeDtypeStruct((NC, n_pad, d), jnp.float32)

    @functools.partial(pl.kernel, mesh=_mesh(), out_type=out_type,
                       scratch_types=scratch)
    def agg_kernel(h_hbm, src_hbm, dst_hbm, cw_hbm, *rest):
        if compute_norm:
            (dinv_hbm, out_hbm, cout_hbm,
             src_v, dst_v, c_v, rows_a, acc, sem_a, dgt_v, sem_c) = rest
        else:
            (out_hbm, src_v, dst_v, c_v, rows_a, acc, sem_a) = rest

        cid = lax.axis_index("c")
        sid = lax.axis_index("s")
        wid = cid * NS + sid

        _zero_rows(rows_a, CHUNK, d)
        for t in range(rpt // CHUNK):
            pltpu.sync_copy(rows_a, acc.at[pl.ds(sid * rpt + t * CHUNK, CHUNK)])
        plsc.subcore_barrier()

        pltpu.sync_copy(src_hbm.at[wid], src_v)
        pltpu.sync_copy(dst_hbm.at[wid], dst_v)
        pltpu.sync_copy(cw_hbm.at[wid], c_v)

        if compute_norm:
            @pl.loop(0, kw)
            def _(k):
                pltpu.async_copy(dinv_hbm.at[dst_v.at[k]], dgt_v, sem_c).wait()
                for i in range(CHUNK // L):
                    sl = pl.ds(i * L, L)
                    c_v[k, sl] = c_v[k, sl] * dgt_v[sl]

            pltpu.sync_copy(c_v, cout_hbm.at[wid])

        def start_gather(buf, sem, k):
            pltpu.async_copy(h_hbm.at[src_v.at[k]], buf, sem)

        def wait_gather(buf, sem):
            pltpu.make_async_copy(h_hbm.at[pl.ds(0, CHUNK)], buf, sem).wait()

        def scale(buf, k):
            @pl.loop(0, CHUNK // L)
            def _(i):
                c16 = c_v[k, pl.ds(i * L, L)]
                for ei in range(L):
                    s = c16[ei]
                    e = i * L + ei
                    for j in range(d // L):
                        sl = pl.ds(j * L, L)
                        buf[e, sl] = buf[e, sl] * s

        def scatter(buf, k):
            pltpu.sync_copy(buf, acc.at[dst_v.at[k]], add=True)

        @pl.loop(0, kw)
        def _(k):
            start_gather(rows_a, sem_a, k)
            wait_gather(rows_a, sem_a)
            scale(rows_a, k)
            scatter(rows_a, k)

        plsc.subcore_barrier()
        pltpu.sync_copy(acc.at[pl.ds(sid * rpt, rpt)],
                        out_hbm.at[cid, pl.ds(sid * rpt, rpt)])

    return agg_kernel


def _tc_prep_body(degp, x, w, dinv_o, h_o):
    deg = degp[0, :] + degp[1, :]
    dinv = jnp.where(deg > 0, lax.rsqrt(jnp.maximum(deg, 1e-12)), 0.0)
    dinv_o[:] = dinv
    h = jnp.dot(x[:, :], w[:, :], preferred_element_type=jnp.float32,
                precision=lax.Precision.HIGHEST)
    h_o[:, :] = h * dinv[:, None]


def _tc_mid_body(p0, p1, b, w, dinv, h_o):
    z = jax.nn.relu(p0[:, :] + p1[:, :] + b[0, :][None, :])
    h = jnp.dot(z, w[:, :], preferred_element_type=jnp.float32,
                precision=lax.Precision.HIGHEST)
    h_o[:, :] = h * dinv[:][:, None]


def _tc_final_body(p0, p1, b, out_o):
    out_o[:, :] = p0[:, :] + p1[:, :] + b[0, :][None, :]


def _tc_prep(degp, x_pad, W1, n_pad, d, bn=1024):
    grid = (n_pad // bn,)
    return pl.pallas_call(
        _tc_prep_body,
        grid=grid,
        in_specs=[
            pl.BlockSpec((NC, bn), lambda i: (0, i)),
            pl.BlockSpec((bn, d), lambda i: (i, 0)),
            pl.BlockSpec((d, d), lambda i: (0, 0)),
        ],
        out_specs=[
            pl.BlockSpec((bn,), lambda i: (i,)),
            pl.BlockSpec((bn, d), lambda i: (i, 0)),
        ],
        out_shape=[
            jax.ShapeDtypeStruct((n_pad,), jnp.float32),
            jax.ShapeDtypeStruct((n_pad, d), jnp.float32),
        ],
    )(degp, x_pad, W1)


def _tc_mid(p0, p1, b1, W2, dinv, n_pad, d, bn=1024):
    grid = (n_pad // bn,)
    return pl.pallas_call(
        _tc_mid_body,
        grid=grid,
        in_specs=[
            pl.BlockSpec((bn, d), lambda i: (i, 0)),
            pl.BlockSpec((bn, d), lambda i: (i, 0)),
            pl.BlockSpec((1, d), lambda i: (0, 0)),
            pl.BlockSpec((d, d), lambda i: (0, 0)),
            pl.BlockSpec((bn,), lambda i: (i,)),
        ],
        out_specs=pl.BlockSpec((bn, d), lambda i: (i, 0)),
        out_shape=jax.ShapeDtypeStruct((n_pad, d), jnp.float32),
    )(p0, p1, b1, W2, dinv)


def _tc_final(p0, p1, b2, n, d, bn=1000):
    grid = (n // bn,)
    return pl.pallas_call(
        _tc_final_body,
        grid=grid,
        in_specs=[
            pl.BlockSpec((bn, d), lambda i: (i, 0)),
            pl.BlockSpec((bn, d), lambda i: (i, 0)),
            pl.BlockSpec((1, d), lambda i: (0, 0)),
        ],
        out_specs=pl.BlockSpec((bn, d), lambda i: (i, 0)),
        out_shape=jax.ShapeDtypeStruct((n, d), jnp.float32),
    )(p0, p1, b2)


def kernel(x, edge_index, edge_weight, W1, b1, W2, b2):
    n, d = x.shape
    e = edge_index.shape[1]
    n_pad = -(-n // (NS * CHUNK)) * (NS * CHUNK)
    kw = -(-e // (NW * CHUNK))
    e_pad = NW * CHUNK * kw

    src = edge_index[0].astype(jnp.int32)
    dst = edge_index[1].astype(jnp.int32)
    w = edge_weight.astype(jnp.float32)
    npad_e = e_pad - e
    if npad_e:
        fill = (jnp.arange(npad_e, dtype=jnp.int32) * 97) % n
        src = jnp.concatenate([src, fill])
        dst = jnp.concatenate([dst, fill])
        w = jnp.concatenate([w, jnp.zeros((npad_e,), jnp.float32)])
    src3 = src.reshape(NW, kw, CHUNK)
    dst3 = dst.reshape(NW, kw, CHUNK)
    w3 = w.reshape(NW, kw, CHUNK)
    x_pad = jnp.pad(x, ((0, n_pad - n), (0, 0)))

    degp = _make_deg_kernel(kw, n_pad)(dst3, w3).reshape(NC, n_pad)
    dinv, h1 = _tc_prep(degp, x_pad, W1, n_pad, d)
    aggp1, c3 = _make_agg_kernel(kw, n_pad, d, True)(h1, src3, dst3, w3, dinv)
    h2 = _tc_mid(aggp1[0], aggp1[1], b1.reshape(1, d), W2, dinv, n_pad, d)
    aggp2 = _make_agg_kernel(kw, n_pad, d, False)(h2, src3, dst3, c3)
    out = _tc_final(aggp2[0], aggp2[1], b2.reshape(1, d), n, d)
    return out


# speedup vs baseline: 19.4825x; 1.4394x over previous
import functools
import math

import jax
import jax.numpy as jnp
from jax import lax
from jax.experimental import pallas as pl
from jax.experimental.pallas import tpu as pltpu
from jax.experimental.pallas import tpu_sc as plsc

NC = 2
NS = 16
NW = NC * NS
L = 16
CHUNK = 128


def _mesh():
    return plsc.VectorSubcoreMesh(core_axis_name="c", subcore_axis_name="s")


def _zero_rows(buf, n_rows, d):
    zv = jnp.zeros((L,), jnp.float32)

    @pl.loop(0, n_rows)
    def _(i):
        for j in range(d // L):
            buf[i, pl.ds(j * L, L)] = zv


def _make_deg_kernel(kw, n_pad):
    rpt = n_pad // NS

    @functools.partial(
        pl.kernel,
        mesh=_mesh(),
        out_type=jax.ShapeDtypeStruct((NC * n_pad,), jnp.float32),
        scratch_types=[
            pltpu.VMEM((kw, CHUNK), jnp.int32),
            pltpu.VMEM((kw, CHUNK), jnp.float32),
            pltpu.VMEM((rpt,), jnp.float32),
            pltpu.VMEM_SHARED((n_pad,), jnp.float32),
        ],
    )
    def deg_kernel(dst_hbm, w_hbm, out_hbm, dst_v, w_v, zb, acc):
        cid = lax.axis_index("c")
        sid = lax.axis_index("s")
        wid = cid * NS + sid

        @pl.loop(0, rpt // L)
        def _(i):
            zb[pl.ds(i * L, L)] = jnp.zeros((L,), jnp.float32)

        pltpu.sync_copy(zb, acc.at[pl.ds(sid * rpt, rpt)])
        plsc.subcore_barrier()

        pltpu.sync_copy(dst_hbm.at[wid], dst_v)
        pltpu.sync_copy(w_hbm.at[wid], w_v)

        @pl.loop(0, kw)
        def _(k):
            pltpu.sync_copy(w_v.at[k], acc.at[dst_v.at[k]], add=True)

        plsc.subcore_barrier()
        pltpu.sync_copy(acc.at[pl.ds(sid * rpt, rpt)],
                        out_hbm.at[pl.ds(cid * n_pad + sid * rpt, rpt)])

    return deg_kernel


def _make_agg_kernel(kw, n_pad, d, compute_norm):
    assert kw % 2 == 0 and kw >= 4
    rpt = n_pad // NS

    scratch = [
        pltpu.VMEM((CHUNK,), jnp.int32),
        pltpu.VMEM((CHUNK,), jnp.int32),
        pltpu.VMEM((CHUNK,), jnp.int32),
        pltpu.VMEM((CHUNK,), jnp.int32),
        pltpu.VMEM((CHUNK,), jnp.float32),
        pltpu.VMEM((CHUNK,), jnp.float32),
        pltpu.VMEM((CHUNK, d), jnp.float32),
        pltpu.VMEM((CHUNK, d), jnp.float32),
        pltpu.VMEM_SHARED((n_pad, d), jnp.float32),
        pltpu.SemaphoreType.DMA,
        pltpu.SemaphoreType.DMA,
        pltpu.SemaphoreType.DMA,
        pltpu.SemaphoreType.DMA,
    ]
    if compute_norm:
        scratch.append(pltpu.VMEM((CHUNK,), jnp.float32))
        scratch.append(pltpu.VMEM((CHUNK,), jnp.float32))
        scratch.append(pltpu.VMEM((kw * CHUNK,), jnp.float32))
        out_type = (jax.ShapeDtypeStruct((NC, n_pad, d), jnp.float32),
                    jax.ShapeDtypeStruct((NW * kw * CHUNK,), jnp.float32))
    else:
        out_type = jax.ShapeDtypeStruct((NC, n_pad, d), jnp.float32)

    @functools.partial(pl.kernel, mesh=_mesh(), out_type=out_type,
                       scratch_types=scratch)
    def agg_kernel(h_hbm, src_hbm, dst_hbm, cw_hbm, *rest):
        if compute_norm:
            (dinv_hbm, out_hbm, cout_hbm, src_a, src_b, dst_a, dst_b,
             cw_a, cw_b, rows_a, rows_b, acc, se_a, se_b, sg_a, sg_b,
             dgt_a, dgt_b, c_v) = rest
            dgtb = (dgt_a, dgt_b)
        else:
            (out_hbm, src_a, src_b, dst_a, dst_b,
             cw_a, cw_b, rows_a, rows_b, acc, se_a, se_b, sg_a, sg_b) = rest

        cid = lax.axis_index("c")
        sid = lax.axis_index("s")
        wid = cid * NS + sid
        ebase = wid * kw * CHUNK

        srcb = (src_a, src_b)
        dstb = (dst_a, dst_b)
        cwb = (cw_a, cw_b)
        rowsb = (rows_a, rows_b)
        seme = (se_a, se_b)
        semg = (sg_a, sg_b)

        _zero_rows(rows_a, CHUNK, d)
        for t in range(rpt // CHUNK):
            pltpu.sync_copy(rows_a, acc.at[pl.ds(sid * rpt + t * CHUNK, CHUNK)])
        plsc.subcore_barrier()

        def start_edges(k, p):
            off = ebase + k * CHUNK
            pltpu.async_copy(src_hbm.at[pl.ds(off, CHUNK)], srcb[p], seme[p])
            pltpu.async_copy(dst_hbm.at[pl.ds(off, CHUNK)], dstb[p], seme[p])
            pltpu.async_copy(cw_hbm.at[pl.ds(off, CHUNK)], cwb[p], seme[p])

        def wait_edges(p):
            pltpu.make_async_copy(src_hbm.at[pl.ds(0, CHUNK)], srcb[p],
                                  seme[p]).wait()
            pltpu.make_async_copy(dst_hbm.at[pl.ds(0, CHUNK)], dstb[p],
                                  seme[p]).wait()
            pltpu.make_async_copy(cw_hbm.at[pl.ds(0, CHUNK)], cwb[p],
                                  seme[p]).wait()

        def start_gathers(p):
            pltpu.async_copy(h_hbm.at[srcb[p]], rowsb[p], semg[p])
            if compute_norm:
                pltpu.async_copy(dinv_hbm.at[dstb[p]], dgtb[p], semg[p])

        def wait_gathers(p):
            pltpu.make_async_copy(h_hbm.at[pl.ds(0, CHUNK)], rowsb[p],
                                  semg[p]).wait()
            if compute_norm:
                pltpu.make_async_copy(cw_hbm.at[pl.ds(0, CHUNK)],
                                      dgtb[p], semg[p]).wait()

        def scale(k, p):
            buf = rowsb[p]
            if compute_norm:
                dgt = dgtb[p]
                for i in range(CHUNK // L):
                    sl = pl.ds(i * L, L)
                    c_v[pl.ds(k * CHUNK + i * L, L)] = cwb[p][sl] * dgt[sl]

            @pl.loop(0, CHUNK // L)
            def _(i):
                sl = pl.ds(i * L, L)
                if compute_norm:
                    c16 = c_v[pl.ds(k * CHUNK + i * L, L)]
                else:
                    c16 = cwb[p][sl]
                for ei in range(L):
                    s = c16[ei]
                    e = i * L + ei
                    for j in range(d // L):
                        slj = pl.ds(j * L, L)
                        buf[e, slj] = buf[e, slj] * s

        def scatter(p):
            pltpu.sync_copy(rowsb[p], acc.at[dstb[p]], add=True)

        def body(k, p):
            q = 1 - p
            wait_edges(q)
            start_gathers(q)
            wait_gathers(p)
            scale(k, p)
            scatter(p)
            start_edges(k + 2, p)

        start_edges(0, 0)
        wait_edges(0)
        start_gathers(0)
        start_edges(1, 1)

        @pl.loop(0, (kw - 2) // 2)
        def _(u):
            body(2 * u, 0)
            body(2 * u + 1, 1)

        wait_edges(1)
        start_gathers(1)
        wait_gathers(0)
        scale(kw - 2, 0)
        scatter(0)
        wait_gathers(1)
        scale(kw - 1, 1)
        scatter(1)

        if compute_norm:
            pltpu.sync_copy(c_v, cout_hbm.at[pl.ds(ebase, kw * CHUNK)])

        plsc.subcore_barrier()
        pltpu.sync_copy(acc.at[pl.ds(sid * rpt, rpt)],
                        out_hbm.at[cid, pl.ds(sid * rpt, rpt)])

    return agg_kernel


def _tc_prep_body(degp, x, w, dinv_o, h_o):
    deg = degp[0, :] + degp[1, :]
    dinv = jnp.where(deg > 0, lax.rsqrt(jnp.maximum(deg, 1e-12)), 0.0)
    dinv_o[:] = dinv
    h = jnp.dot(x[:, :], w[:, :], preferred_element_type=jnp.float32,
                precision=lax.Precision.HIGHEST)
    h_o[:, :] = h * dinv[:, None]


def _tc_mid_body(p0, p1, b, w, dinv, h_o):
    z = jax.nn.relu(p0[:, :] + p1[:, :] + b[0, :][None, :])
    h = jnp.dot(z, w[:, :], preferred_element_type=jnp.float32,
                precision=lax.Precision.HIGHEST)
    h_o[:, :] = h * dinv[:][:, None]


def _tc_final_body(p0, p1, b, out_o):
    out_o[:, :] = p0[:, :] + p1[:, :] + b[0, :][None, :]


def _tc_prep(degp, x_pad, W1, n_pad, d, bn=1024):
    grid = (n_pad // bn,)
    return pl.pallas_call(
        _tc_prep_body,
        grid=grid,
        in_specs=[
            pl.BlockSpec((NC, bn), lambda i: (0, i)),
            pl.BlockSpec((bn, d), lambda i: (i, 0)),
            pl.BlockSpec((d, d), lambda i: (0, 0)),
        ],
        out_specs=[
            pl.BlockSpec((bn,), lambda i: (i,)),
            pl.BlockSpec((bn, d), lambda i: (i, 0)),
        ],
        out_shape=[
            jax.ShapeDtypeStruct((n_pad,), jnp.float32),
            jax.ShapeDtypeStruct((n_pad, d), jnp.float32),
        ],
    )(degp, x_pad, W1)


def _tc_mid(p0, p1, b1, W2, dinv, n_pad, d, bn=1024):
    grid = (n_pad // bn,)
    return pl.pallas_call(
        _tc_mid_body,
        grid=grid,
        in_specs=[
            pl.BlockSpec((bn, d), lambda i: (i, 0)),
            pl.BlockSpec((bn, d), lambda i: (i, 0)),
            pl.BlockSpec((1, d), lambda i: (0, 0)),
            pl.BlockSpec((d, d), lambda i: (0, 0)),
            pl.BlockSpec((bn,), lambda i: (i,)),
        ],
        out_specs=pl.BlockSpec((bn, d), lambda i: (i, 0)),
        out_shape=jax.ShapeDtypeStruct((n_pad, d), jnp.float32),
    )(p0, p1, b1, W2, dinv)


def _tc_final(p0, p1, b2, n, d, bn=1000):
    grid = (n // bn,)
    return pl.pallas_call(
        _tc_final_body,
        grid=grid,
        in_specs=[
            pl.BlockSpec((bn, d), lambda i: (i, 0)),
            pl.BlockSpec((bn, d), lambda i: (i, 0)),
            pl.BlockSpec((1, d), lambda i: (0, 0)),
        ],
        out_specs=pl.BlockSpec((bn, d), lambda i: (i, 0)),
        out_shape=jax.ShapeDtypeStruct((n, d), jnp.float32),
    )(p0, p1, b2)


def kernel(x, edge_index, edge_weight, W1, b1, W2, b2):
    n, d = x.shape
    e = edge_index.shape[1]
    n_pad = -(-n // (NS * CHUNK)) * (NS * CHUNK)
    kw = -(-e // (NW * CHUNK))
    kw = kw + (kw % 2)
    e_pad = NW * CHUNK * kw

    src = edge_index[0].astype(jnp.int32)
    dst = edge_index[1].astype(jnp.int32)
    w = edge_weight.astype(jnp.float32)
    npad_e = e_pad - e
    if npad_e:
        fill = (jnp.arange(npad_e, dtype=jnp.int32) * 97) % n
        src = jnp.concatenate([src, fill])
        dst = jnp.concatenate([dst, fill])
        w = jnp.concatenate([w, jnp.zeros((npad_e,), jnp.float32)])
    dst3 = dst.reshape(NW, kw, CHUNK)
    w3 = w.reshape(NW, kw, CHUNK)
    x_pad = jnp.pad(x, ((0, n_pad - n), (0, 0)))

    degp = _make_deg_kernel(kw, n_pad)(dst3, w3).reshape(NC, n_pad)
    dinv, h1 = _tc_prep(degp, x_pad, W1, n_pad, d)
    aggp1, cflat = _make_agg_kernel(kw, n_pad, d, True)(h1, src, dst, w, dinv)
    h2 = _tc_mid(aggp1[0], aggp1[1], b1.reshape(1, d), W2, dinv, n_pad, d)
    aggp2 = _make_agg_kernel(kw, n_pad, d, False)(h2, src, dst, cflat)
    out = _tc_final(aggp2[0], aggp2[1], b2.reshape(1, d), n, d)
    return out


# speedup vs baseline: 22.3453x; 1.1469x over previous
import functools

import jax
import jax.numpy as jnp
from jax import lax
from jax.experimental import pallas as pl
from jax.experimental.pallas import tpu as pltpu
from jax.experimental.pallas import tpu_sc as plsc

NC = 2
NS = 16
NW = NC * NS
L = 16
CHUNK = 128


def _mesh():
    return plsc.VectorSubcoreMesh(core_axis_name="c", subcore_axis_name="s")


def _zero_rows(buf, n_rows, d):
    zv = jnp.zeros((L,), jnp.float32)

    @pl.loop(0, n_rows)
    def _(i):
        for j in range(d // L):
            buf[i, pl.ds(j * L, L)] = zv


def _make_deg_kernel(kw, n_pad):
    rpt = n_pad // NS

    @functools.partial(
        pl.kernel,
        mesh=_mesh(),
        out_type=jax.ShapeDtypeStruct((NC * n_pad,), jnp.float32),
        scratch_types=[
            pltpu.VMEM((kw, CHUNK), jnp.int32),
            pltpu.VMEM((kw, CHUNK), jnp.float32),
            pltpu.VMEM((rpt,), jnp.float32),
            pltpu.VMEM_SHARED((n_pad,), jnp.float32),
        ],
    )
    def deg_kernel(dst_hbm, w_hbm, out_hbm, dst_v, w_v, zb, acc):
        cid = lax.axis_index("c")
        sid = lax.axis_index("s")
        wid = cid * NS + sid

        @pl.loop(0, rpt // L)
        def _(i):
            zb[pl.ds(i * L, L)] = jnp.zeros((L,), jnp.float32)

        pltpu.sync_copy(zb, acc.at[pl.ds(sid * rpt, rpt)])
        plsc.subcore_barrier()

        pltpu.sync_copy(dst_hbm.at[wid], dst_v)
        pltpu.sync_copy(w_hbm.at[wid], w_v)

        @pl.loop(0, kw)
        def _(k):
            pltpu.sync_copy(w_v.at[k], acc.at[dst_v.at[k]], add=True)

        plsc.subcore_barrier()
        pltpu.sync_copy(acc.at[pl.ds(sid * rpt, rpt)],
                        out_hbm.at[pl.ds(cid * n_pad + sid * rpt, rpt)])

    return deg_kernel


def _make_agg_kernel(kw, n_pad, d, compute_norm):
    assert kw % 2 == 0 and kw >= 4
    rpt = n_pad // NS

    scratch = [
        pltpu.VMEM((CHUNK,), jnp.int32),
        pltpu.VMEM((CHUNK,), jnp.int32),
        pltpu.VMEM((CHUNK,), jnp.int32),
        pltpu.VMEM((CHUNK,), jnp.int32),
        pltpu.VMEM((CHUNK,), jnp.float32),
        pltpu.VMEM((CHUNK,), jnp.float32),
        pltpu.VMEM((CHUNK, d), jnp.float32),
        pltpu.VMEM((CHUNK, d), jnp.float32),
        pltpu.VMEM_SHARED((n_pad, d), jnp.float32),
        pltpu.SemaphoreType.DMA,
        pltpu.SemaphoreType.DMA,
        pltpu.SemaphoreType.DMA,
        pltpu.SemaphoreType.DMA,
        pltpu.SemaphoreType.DMA,
        pltpu.SemaphoreType.DMA,
    ]
    if compute_norm:
        scratch.append(pltpu.VMEM((CHUNK,), jnp.float32))
        scratch.append(pltpu.VMEM((CHUNK,), jnp.float32))
        scratch.append(pltpu.VMEM((kw * CHUNK,), jnp.float32))
        out_type = (jax.ShapeDtypeStruct((NC, n_pad, d), jnp.float32),
                    jax.ShapeDtypeStruct((NW * kw * CHUNK,), jnp.float32))
    else:
        out_type = jax.ShapeDtypeStruct((NC, n_pad, d), jnp.float32)

    @functools.partial(pl.kernel, mesh=_mesh(), out_type=out_type,
                       scratch_types=scratch)
    def agg_kernel(h_hbm, src_hbm, dst_hbm, cw_hbm, *rest):
        if compute_norm:
            (dinv_hbm, out_hbm, cout_hbm, src_a, src_b, dst_a, dst_b,
             cw_a, cw_b, rows_a, rows_b, acc, se_a, se_b, sg_a, sg_b,
             ss_a, ss_b, dgt_a, dgt_b, c_v) = rest
            dgtb = (dgt_a, dgt_b)
        else:
            (out_hbm, src_a, src_b, dst_a, dst_b,
             cw_a, cw_b, rows_a, rows_b, acc,
             se_a, se_b, sg_a, sg_b, ss_a, ss_b) = rest

        cid = lax.axis_index("c")
        sid = lax.axis_index("s")
        wid = cid * NS + sid
        ebase = wid * kw * CHUNK

        srcb = (src_a, src_b)
        dstb = (dst_a, dst_b)
        cwb = (cw_a, cw_b)
        rowsb = (rows_a, rows_b)
        seme = (se_a, se_b)
        semg = (sg_a, sg_b)
        sems = (ss_a, ss_b)

        _zero_rows(rows_a, CHUNK, d)
        for t in range(rpt // CHUNK):
            pltpu.sync_copy(rows_a, acc.at[pl.ds(sid * rpt + t * CHUNK, CHUNK)])
        plsc.subcore_barrier()

        def start_edges(k, p):
            off = ebase + k * CHUNK
            pltpu.async_copy(src_hbm.at[pl.ds(off, CHUNK)], srcb[p], seme[p])
            pltpu.async_copy(dst_hbm.at[pl.ds(off, CHUNK)], dstb[p], seme[p])
            pltpu.async_copy(cw_hbm.at[pl.ds(off, CHUNK)], cwb[p], seme[p])

        def wait_edges(p):
            pltpu.make_async_copy(src_hbm.at[pl.ds(0, CHUNK)], srcb[p],
                                  seme[p]).wait()
            pltpu.make_async_copy(dst_hbm.at[pl.ds(0, CHUNK)], dstb[p],
                                  seme[p]).wait()
            pltpu.make_async_copy(cw_hbm.at[pl.ds(0, CHUNK)], cwb[p],
                                  seme[p]).wait()

        def start_gathers(p):
            pltpu.async_copy(h_hbm.at[srcb[p]], rowsb[p], semg[p])
            if compute_norm:
                pltpu.async_copy(dinv_hbm.at[dstb[p]], dgtb[p], semg[p])

        def wait_gathers(p):
            pltpu.make_async_copy(h_hbm.at[pl.ds(0, CHUNK)], rowsb[p],
                                  semg[p]).wait()
            if compute_norm:
                pltpu.make_async_copy(cw_hbm.at[pl.ds(0, CHUNK)],
                                      dgtb[p], semg[p]).wait()

        def scale(k, p):
            buf = rowsb[p]
            if compute_norm:
                dgt = dgtb[p]
                for i in range(CHUNK // L):
                    sl = pl.ds(i * L, L)
                    c_v[pl.ds(k * CHUNK + i * L, L)] = cwb[p][sl] * dgt[sl]

            @pl.loop(0, CHUNK // L, unroll=2)
            def _(i):
                sl = pl.ds(i * L, L)
                if compute_norm:
                    c16 = c_v[pl.ds(k * CHUNK + i * L, L)]
                else:
                    c16 = cwb[p][sl]
                for ei in range(L):
                    s = c16[ei]
                    e = i * L + ei
                    for j in range(d // L):
                        slj = pl.ds(j * L, L)
                        buf[e, slj] = buf[e, slj] * s

        def start_scatter(p):
            pltpu.async_copy(rowsb[p], acc.at[dstb[p]], sems[p], add=True)

        def wait_scatter(p):
            pltpu.make_async_copy(h_hbm.at[pl.ds(0, CHUNK)], rowsb[p],
                                  sems[p]).wait()

        def body(k, p, first=False):
            q = 1 - p
            wait_edges(q)
            if not first:
                wait_scatter(q)
            start_gathers(q)
            wait_gathers(p)
            scale(k, p)
            start_scatter(p)
            start_edges(k + 2, p)

        start_edges(0, 0)
        wait_edges(0)
        start_gathers(0)
        start_edges(1, 1)

        body(0, 0, first=True)
        body(1, 1)

        @pl.loop(0, (kw - 4) // 2)
        def _(u):
            body(2 * u + 2, 0)
            body(2 * u + 3, 1)

        wait_edges(1)
        wait_scatter(1)
        start_gathers(1)
        wait_gathers(0)
        scale(kw - 2, 0)
        start_scatter(0)
        wait_gathers(1)
        scale(kw - 1, 1)
        start_scatter(1)
        wait_scatter(0)
        wait_scatter(1)

        if compute_norm:
            pltpu.sync_copy(c_v, cout_hbm.at[pl.ds(ebase, kw * CHUNK)])

        plsc.subcore_barrier()
        pltpu.sync_copy(acc.at[pl.ds(sid * rpt, rpt)],
                        out_hbm.at[cid, pl.ds(sid * rpt, rpt)])

    return agg_kernel


def _tc_prep_body(degp, x, w, dinv_o, h_o):
    deg = degp[0, :] + degp[1, :]
    dinv = jnp.where(deg > 0, lax.rsqrt(jnp.maximum(deg, 1e-12)), 0.0)
    dinv_o[:] = dinv
    h = jnp.dot(x[:, :], w[:, :], preferred_element_type=jnp.float32,
                precision=lax.Precision.HIGHEST)
    h_o[:, :] = h * dinv[:, None]


def _tc_mid_body(p0, p1, b, w, dinv, h_o):
    z = jax.nn.relu(p0[:, :] + p1[:, :] + b[0, :][None, :])
    h = jnp.dot(z, w[:, :], preferred_element_type=jnp.float32,
                precision=lax.Precision.HIGHEST)
    h_o[:, :] = h * dinv[:][:, None]


def _tc_final_body(p0, p1, b, out_o):
    out_o[:, :] = p0[:, :] + p1[:, :] + b[0, :][None, :]


def _tc_prep(degp, x_pad, W1, n_pad, d, bn=1024):
    grid = (n_pad // bn,)
    return pl.pallas_call(
        _tc_prep_body,
        grid=grid,
        in_specs=[
            pl.BlockSpec((NC, bn), lambda i: (0, i)),
            pl.BlockSpec((bn, d), lambda i: (i, 0)),
            pl.BlockSpec((d, d), lambda i: (0, 0)),
        ],
        out_specs=[
            pl.BlockSpec((bn,), lambda i: (i,)),
            pl.BlockSpec((bn, d), lambda i: (i, 0)),
        ],
        out_shape=[
            jax.ShapeDtypeStruct((n_pad,), jnp.float32),
            jax.ShapeDtypeStruct((n_pad, d), jnp.float32),
        ],
    )(degp, x_pad, W1)


def _tc_mid(p0, p1, b1, W2, dinv, n_pad, d, bn=1024):
    grid = (n_pad // bn,)
    return pl.pallas_call(
        _tc_mid_body,
        grid=grid,
        in_specs=[
            pl.BlockSpec((bn, d), lambda i: (i, 0)),
            pl.BlockSpec((bn, d), lambda i: (i, 0)),
            pl.BlockSpec((1, d), lambda i: (0, 0)),
            pl.BlockSpec((d, d), lambda i: (0, 0)),
            pl.BlockSpec((bn,), lambda i: (i,)),
        ],
        out_specs=pl.BlockSpec((bn, d), lambda i: (i, 0)),
        out_shape=jax.ShapeDtypeStruct((n_pad, d), jnp.float32),
    )(p0, p1, b1, W2, dinv)


def _tc_final(p0, p1, b2, n, d, bn=1000):
    grid = (n // bn,)
    return pl.pallas_call(
        _tc_final_body,
        grid=grid,
        in_specs=[
            pl.BlockSpec((bn, d), lambda i: (i, 0)),
            pl.BlockSpec((bn, d), lambda i: (i, 0)),
            pl.BlockSpec((1, d), lambda i: (0, 0)),
        ],
        out_specs=pl.BlockSpec((bn, d), lambda i: (i, 0)),
        out_shape=jax.ShapeDtypeStruct((n, d), jnp.float32),
    )(p0, p1, b2)


def kernel(x, edge_index, edge_weight, W1, b1, W2, b2):
    n, d = x.shape
    e = edge_index.shape[1]
    n_pad = -(-n // (NS * CHUNK)) * (NS * CHUNK)
    kw = -(-e // (NW * CHUNK))
    kw = kw + (kw % 2)

    src = edge_index[0].astype(jnp.int32)
    dst = edge_index[1].astype(jnp.int32)
    w = edge_weight.astype(jnp.float32)
    e_pad = NW * CHUNK * kw
    npad_e = e_pad - e
    if npad_e:
        fill = (jnp.arange(npad_e, dtype=jnp.int32) * 97) % n
        src = jnp.concatenate([src, fill])
        dst = jnp.concatenate([dst, fill])
        w = jnp.concatenate([w, jnp.zeros((npad_e,), jnp.float32)])
    dst3 = dst.reshape(NW, kw, CHUNK)
    w3 = w.reshape(NW, kw, CHUNK)
    x_pad = jnp.pad(x, ((0, n_pad - n), (0, 0)))

    degp = _make_deg_kernel(kw, n_pad)(dst3, w3).reshape(NC, n_pad)
    dinv, h1 = _tc_prep(degp, x_pad, W1, n_pad, d)
    aggp1, cflat = _make_agg_kernel(kw, n_pad, d, True)(h1, src, dst, w, dinv)
    h2 = _tc_mid(aggp1[0], aggp1[1], b1.reshape(1, d), W2, dinv, n_pad, d)
    aggp2 = _make_agg_kernel(kw, n_pad, d, False)(h2, src, dst, cflat)
    out = _tc_final(aggp2[0], aggp2[1], b2.reshape(1, d), n, d)
    return out
